# baseline (speedup 1.0000x reference)
import functools

import jax
import jax.numpy as jnp
from jax import lax
from jax.experimental import pallas as pl
from jax.experimental.pallas import tpu as pltpu
from jax.experimental.pallas import tpu_sc as plsc

B = 16384
D = 32
R = 4
NC = 2
NS = 16
L = 16
NW = NC * NS
BPW = B // NW
C = 64
NCH = BPW // C

_mesh = plsc.VectorSubcoreMesh(core_axis_name="c", subcore_axis_name="s")


@functools.partial(
    pl.kernel,
    out_type=(jax.ShapeDtypeStruct((B,), jnp.float32),
              jax.ShapeDtypeStruct((B,), jnp.float32)),
    mesh=_mesh,
    compiler_params=pltpu.CompilerParams(needs_layout_passes=False),
    scratch_types=(
        [pltpu.VMEM((C,), jnp.int32) for _ in range(5)]
        + [pltpu.VMEM((C,), jnp.int32) for _ in range(5)]
        + [pltpu.VMEM((C, D * R), jnp.float32) for _ in range(10)]
        + [pltpu.VMEM((BPW,), jnp.float32) for _ in range(2)]
        + [pltpu.SemaphoreType.DMA]
    ),
)
def _complex_score_sc(h, t, nh, nt, r, re_ent, im_ent, re_rel, im_rel,
                      pos_out, neg_out,
                      hc, tc, nhc, ntc, rc,
                      hdc, tdc, nhdc, ntdc, rdc,
                      reh_v, imh_v, ret_v, imt_v,
                      renh_v, imnh_v, rent_v, imnt_v,
                      rer_v, imr_v,
                      pos_v, neg_v, sem):
    wid = lax.axis_index("s") * NC + lax.axis_index("c")
    base = wid * BPW

    def chunk_body(c, _):
        cb = c * C
        idx_descs = [
            pltpu.async_copy(h.at[pl.ds(base + cb, C)], hc, sem),
            pltpu.async_copy(t.at[pl.ds(base + cb, C)], tc, sem),
            pltpu.async_copy(nh.at[pl.ds(base + cb, C)], nhc, sem),
            pltpu.async_copy(nt.at[pl.ds(base + cb, C)], ntc, sem),
            pltpu.async_copy(r.at[pl.ds(base + cb, C)], rc, sem),
        ]
        for dsc in idx_descs:
            dsc.wait()
        for j in range(C // L):
            sl = pl.ds(j * L, L)
            hdc[sl] = hc[sl] // R
            tdc[sl] = tc[sl] // R
            nhdc[sl] = nhc[sl] // R
            ntdc[sl] = ntc[sl] // R
            rdc[sl] = rc[sl] // R
        descs = [
            pltpu.async_copy(re_ent.at[hdc], reh_v, sem),
            pltpu.async_copy(im_ent.at[hdc], imh_v, sem),
            pltpu.async_copy(re_ent.at[tdc], ret_v, sem),
            pltpu.async_copy(im_ent.at[tdc], imt_v, sem),
            pltpu.async_copy(re_ent.at[nhdc], renh_v, sem),
            pltpu.async_copy(im_ent.at[nhdc], imnh_v, sem),
            pltpu.async_copy(re_ent.at[ntdc], rent_v, sem),
            pltpu.async_copy(im_ent.at[ntdc], imnt_v, sem),
            pltpu.async_copy(re_rel.at[rdc], rer_v, sem),
            pltpu.async_copy(im_rel.at[rdc], imr_v, sem),
        ]
        for dsc in descs:
            dsc.wait()

        def g_body(g, _):
            gl = pl.ds(g * L, L)
            elem = lax.iota(jnp.int32, L) + g * L
            sub_h = (hc[gl] & (R - 1)) * D
            sub_t = (tc[gl] & (R - 1)) * D
            sub_nh = (nhc[gl] & (R - 1)) * D
            sub_nt = (ntc[gl] & (R - 1)) * D
            sub_r = (rc[gl] & (R - 1)) * D
            accp = jnp.zeros((L,), jnp.float32)
            accn = jnp.zeros((L,), jnp.float32)
            for d in range(D):
                ih = [elem, sub_h + d]
                it = [elem, sub_t + d]
                inh = [elem, sub_nh + d]
                int_ = [elem, sub_nt + d]
                ir = [elem, sub_r + d]
                reh = plsc.load_gather(reh_v, ih)
                imh = plsc.load_gather(imh_v, ih)
                ret = plsc.load_gather(ret_v, it)
                imt = plsc.load_gather(imt_v, it)
                renh = plsc.load_gather(renh_v, inh)
                imnh = plsc.load_gather(imnh_v, inh)
                rent = plsc.load_gather(rent_v, int_)
                imnt = plsc.load_gather(imnt_v, int_)
                rer = plsc.load_gather(rer_v, ir)
                imr = plsc.load_gather(imr_v, ir)
                accp = accp + rer * (reh * ret + imh * imt)
                accp = accp + imr * ((reh - imh) * imt)
                accn = accn + rer * (renh * rent + imnh * imnt)
                accn = accn + imr * ((renh - imnh) * imnt)
            pos_v[pl.ds(cb + g * L, L)] = accp
            neg_v[pl.ds(cb + g * L, L)] = accn
            return 0

        lax.fori_loop(0, C // L, g_body, 0)
        return 0

    lax.fori_loop(0, NCH, chunk_body, 0)

    pltpu.sync_copy(pos_v, pos_out.at[pl.ds(base, BPW)])
    pltpu.sync_copy(neg_v, neg_out.at[pl.ds(base, BPW)])


def kernel(h, t, nh, nt, r, re_ent, im_ent, re_rel, im_rel):
    n_ent = re_ent.shape[0]
    n_rel = re_rel.shape[0]
    return _complex_score_sc(h.astype(jnp.int32), t.astype(jnp.int32),
                             nh.astype(jnp.int32), nt.astype(jnp.int32),
                             r.astype(jnp.int32),
                             re_ent.reshape(n_ent // R, D * R),
                             im_ent.reshape(n_ent // R, D * R),
                             re_rel.reshape(n_rel // R, D * R),
                             im_rel.reshape(n_rel // R, D * R))
